# baseline (speedup 1.0000x reference)
import functools

import jax
import jax.numpy as jnp
from jax import lax
from jax.experimental import pallas as pl
from jax.experimental.pallas import tpu as pltpu
from jax.experimental.pallas import tpu_sc as plsc

_N = 2000000
_LANES = 16
_VECS = _N // _LANES
_CHUNK_V = 250
_CHUNK_E = _CHUNK_V * _LANES
_NCHUNKS = _VECS // _CHUNK_V
_NW = 32
_MAXC = -(-_NCHUNKS // _NW)


def _sc_body(x_hbm, t_hbm, out_hbm,
             xb0, xb1, tb0, tb1, accv,
             sx0, sx1, st0, st1):
    wid = lax.axis_index("s") * 2 + lax.axis_index("c")
    nchunks_w = (_NCHUNKS - 1 - wid) // _NW + 1

    xbufs = (xb0, xb1)
    tbufs = (tb0, tb1)
    sxs = (sx0, sx1)
    sts = (st0, st1)

    two_iota = lax.iota(jnp.int32, _LANES) * 2

    def x_slice(j):
        return x_hbm.at[pl.ds(j * (2 * _CHUNK_E), 2 * _CHUNK_E)]

    def t_slice(j):
        return t_hbm.at[pl.ds(j * _CHUNK_E, _CHUNK_E)]

    def start(i, b):
        j = wid + i * _NW
        pltpu.async_copy(x_slice(j), xbufs[b], sxs[b])
        pltpu.async_copy(t_slice(j), tbufs[b], sts[b])

    def wait(i, b):
        j = wid + i * _NW
        pltpu.make_async_copy(x_slice(j), xbufs[b], sxs[b]).wait()
        pltpu.make_async_copy(t_slice(j), tbufs[b], sts[b]).wait()

    accv[...] = jnp.zeros((_LANES,), jnp.float32)

    start(0, 0)
    start(1, 1)

    c3 = jnp.float32(2.0 / 3.0)
    c5 = jnp.float32(2.0 / 5.0)
    c7 = jnp.float32(2.0 / 7.0)
    c9 = jnp.float32(2.0 / 9.0)

    def compute_chunk(xbuf, tbuf):
        def vec_body(k, acc):
            i0 = two_iota + k * (2 * _LANES)
            i1 = i0 + 1
            x0 = plsc.load_gather(xbuf, [i0])
            x1 = plsc.load_gather(xbuf, [i1])
            t = tbuf[pl.ds(k * _LANES, _LANES)]
            m = jnp.maximum(x0, x1)
            d = jnp.minimum(x0, x1) - m
            z = jnp.exp(d)
            s = z / (z + jnp.float32(2.0))
            s2 = s * s
            p = jnp.float32(2.0) + s2 * (c3 + s2 * (c5 + s2 * (c7 + s2 * c9)))
            lse = m + s * p
            xt = jnp.where(t == 1, x1, x0)
            return acc + (lse - xt)

        return lax.fori_loop(0, _CHUNK_V, vec_body,
                             jnp.zeros((_LANES,), jnp.float32),
                             unroll=5)

    def outer(u, carry):
        for b in range(2):
            i = u * 2 + b

            @pl.when(i < nchunks_w)
            def _():
                wait(i, b)
                accv[...] += compute_chunk(xbufs[b], tbufs[b])

                @pl.when(i + 2 < nchunks_w)
                def _():
                    start(i + 2, b)
        return carry

    lax.fori_loop(0, _MAXC // 2, outer, 0)

    pltpu.sync_copy(accv, out_hbm.at[wid])


@jax.jit
def _rpn_cls_loss(x_flat, t_flat):
    mesh = plsc.VectorSubcoreMesh(core_axis_name="c", subcore_axis_name="s")
    partials = pl.kernel(
        _sc_body,
        out_type=jax.ShapeDtypeStruct((_NW, _LANES), jnp.float32),
        mesh=mesh,
        compiler_params=pltpu.CompilerParams(needs_layout_passes=False),
        scratch_types=[
            pltpu.VMEM((2 * _CHUNK_E,), jnp.float32),
            pltpu.VMEM((2 * _CHUNK_E,), jnp.float32),
            pltpu.VMEM((_CHUNK_E,), jnp.int32),
            pltpu.VMEM((_CHUNK_E,), jnp.int32),
            pltpu.VMEM((_LANES,), jnp.float32),
            pltpu.SemaphoreType.DMA,
            pltpu.SemaphoreType.DMA,
            pltpu.SemaphoreType.DMA,
            pltpu.SemaphoreType.DMA,
        ],
    )(x_flat, t_flat)
    loss = jnp.sum(partials) * jnp.float32(1.0 / _N)
    return jnp.clip(loss, 0.0, 10.0)


def kernel(input, target):
    x_flat = input.reshape(2 * _N)
    t_flat = target.reshape(_N).astype(jnp.int32)
    return _rpn_cls_loss(x_flat, t_flat)


# speedup vs baseline: 54.5168x; 54.5168x over previous
import functools

import jax
import jax.numpy as jnp
from jax import lax
from jax.experimental import pallas as pl
from jax.experimental.pallas import tpu as pltpu
from jax.experimental.pallas import tpu_sc as plsc

_N = 2000000
_LANES = 16
_BLK = 128
_NBLK = _N // _BLK
_CHUNK_B = 125
_CHUNK_E = _CHUNK_B * _BLK
_CHUNK_V = _CHUNK_E // _LANES
_NCHUNKS = _NBLK // _CHUNK_B
_NW = 32
_MAXC = -(-_NCHUNKS // _NW)


def _sc_body(x_hbm, t_hbm, out_hbm,
             xb0, xb1, tb0, tb1, accv,
             sx0, sx1, st0, st1):
    wid = lax.axis_index("s") * 2 + lax.axis_index("c")
    nchunks_w = (_NCHUNKS - 1 - wid) // _NW + 1

    xbufs = (xb0, xb1)
    tbufs = (tb0, tb1)
    sxs = (sx0, sx1)
    sts = (st0, st1)

    def x_slice(j):
        return x_hbm.at[pl.ds(j * _CHUNK_B, _CHUNK_B), :, :]

    def t_slice(j):
        return t_hbm.at[0, 0, pl.ds(j * _CHUNK_E, _CHUNK_E)]

    def start(i, b):
        j = wid + i * _NW
        pltpu.async_copy(x_slice(j), xbufs[b], sxs[b])
        pltpu.async_copy(t_slice(j), tbufs[b], sts[b])

    def wait(i, b):
        j = wid + i * _NW
        pltpu.make_async_copy(x_slice(j), xbufs[b], sxs[b]).wait()
        pltpu.make_async_copy(t_slice(j), tbufs[b], sts[b]).wait()

    accv[...] = jnp.zeros((_LANES,), jnp.float32)

    start(0, 0)
    start(1, 1)

    c3 = jnp.float32(2.0 / 3.0)
    c5 = jnp.float32(2.0 / 5.0)
    c7 = jnp.float32(2.0 / 7.0)
    c9 = jnp.float32(2.0 / 9.0)

    def compute_chunk(xbuf, tbuf):
        def vec_body(k, acc):
            blk = k // (_BLK // _LANES)
            lane = (k % (_BLK // _LANES)) * _LANES
            x0 = xbuf[blk, 0, pl.ds(lane, _LANES)]
            x1 = xbuf[blk, 1, pl.ds(lane, _LANES)]
            t = tbuf[pl.ds(k * _LANES, _LANES)]
            m = jnp.maximum(x0, x1)
            d = jnp.minimum(x0, x1) - m
            z = jnp.exp(d)
            s = z / (z + jnp.float32(2.0))
            s2 = s * s
            p = jnp.float32(2.0) + s2 * (c3 + s2 * (c5 + s2 * (c7 + s2 * c9)))
            lse = m + s * p
            xt = jnp.where(t == 1, x1, x0)
            return acc + (lse - xt)

        return lax.fori_loop(0, _CHUNK_V, vec_body,
                             jnp.zeros((_LANES,), jnp.float32),
                             unroll=8)

    def outer(u, carry):
        for b in range(2):
            i = u * 2 + b

            @pl.when(i < nchunks_w)
            def _():
                wait(i, b)
                accv[...] += compute_chunk(xbufs[b], tbufs[b])

                @pl.when(i + 2 < nchunks_w)
                def _():
                    start(i + 2, b)
        return carry

    lax.fori_loop(0, _MAXC // 2, outer, 0)

    pltpu.sync_copy(accv, out_hbm.at[wid])


@jax.jit
def _rpn_cls_loss(x_in, t_in):
    x_blk = jnp.transpose(x_in.reshape(_NBLK, _BLK, 2), (0, 2, 1))
    mesh = plsc.VectorSubcoreMesh(core_axis_name="c", subcore_axis_name="s")
    partials = pl.kernel(
        _sc_body,
        out_type=jax.ShapeDtypeStruct((_NW, _LANES), jnp.float32),
        mesh=mesh,
        compiler_params=pltpu.CompilerParams(needs_layout_passes=False),
        scratch_types=[
            pltpu.VMEM((_CHUNK_B, 2, _BLK), jnp.float32),
            pltpu.VMEM((_CHUNK_B, 2, _BLK), jnp.float32),
            pltpu.VMEM((_CHUNK_E,), jnp.int32),
            pltpu.VMEM((_CHUNK_E,), jnp.int32),
            pltpu.VMEM((_LANES,), jnp.float32),
            pltpu.SemaphoreType.DMA,
            pltpu.SemaphoreType.DMA,
            pltpu.SemaphoreType.DMA,
            pltpu.SemaphoreType.DMA,
        ],
    )(x_blk, t_in)
    loss = jnp.sum(partials) * jnp.float32(1.0 / _N)
    return jnp.clip(loss, 0.0, 10.0)


def kernel(input, target):
    return _rpn_cls_loss(input, target)


# speedup vs baseline: 63.3909x; 1.1628x over previous
import functools

import jax
import jax.numpy as jnp
from jax import lax
from jax.experimental import pallas as pl
from jax.experimental.pallas import tpu as pltpu
from jax.experimental.pallas import tpu_sc as plsc

_N = 2000000
_LANES = 16
_BLK = 128
_NBLK = _N // _BLK
_CHUNK_B = 125
_CHUNK_E = _CHUNK_B * _BLK
_CHUNK_V = _CHUNK_E // _LANES
_NCHUNKS = _NBLK // _CHUNK_B
_NW = 32
_MAXC = -(-_NCHUNKS // _NW)


def _sc_body(x_hbm, t_hbm, out_hbm,
             xb0, xb1, tb0, tb1, accv,
             sx0, sx1, st0, st1):
    wid = lax.axis_index("s") * 2 + lax.axis_index("c")
    nchunks_w = (_NCHUNKS - 1 - wid) // _NW + 1

    xbufs = (xb0, xb1)
    tbufs = (tb0, tb1)
    sxs = (sx0, sx1)
    sts = (st0, st1)

    def x_slice(j):
        return x_hbm.at[pl.ds(j * _CHUNK_B, _CHUNK_B), :, :]

    def t_slice(j):
        return t_hbm.at[0, 0, pl.ds(j * _CHUNK_E, _CHUNK_E)]

    def start(i, b):
        j = wid + i * _NW
        pltpu.async_copy(x_slice(j), xbufs[b], sxs[b])
        pltpu.async_copy(t_slice(j), tbufs[b], sts[b])

    def wait(i, b):
        j = wid + i * _NW
        pltpu.make_async_copy(x_slice(j), xbufs[b], sxs[b]).wait()
        pltpu.make_async_copy(t_slice(j), tbufs[b], sts[b]).wait()

    accv[...] = jnp.zeros((_LANES,), jnp.float32)

    start(0, 0)
    start(1, 1)

    c0 = jnp.float32(2.000009811424984)
    c1 = jnp.float32(0.664841814799871)
    c2 = jnp.float32(0.447722291084453)

    def compute_chunk(xbuf, tbuf):
        def vec_body(k, acc):
            blk = k // (_BLK // _LANES)
            lane = (k % (_BLK // _LANES)) * _LANES
            x0 = xbuf[blk, 0, pl.ds(lane, _LANES)]
            x1 = xbuf[blk, 1, pl.ds(lane, _LANES)]
            t = tbuf[pl.ds(k * _LANES, _LANES)]
            m = jnp.maximum(x0, x1)
            d = jnp.minimum(x0, x1) - m
            z = jnp.exp(d)
            s = z / (z + jnp.float32(2.0))
            s2 = s * s
            p = c0 + s2 * (c1 + s2 * c2)
            lse = m + s * p
            xt = jnp.where(t == 1, x1, x0)
            return acc + (lse - xt)

        return lax.fori_loop(0, _CHUNK_V, vec_body,
                             jnp.zeros((_LANES,), jnp.float32),
                             unroll=4)

    def outer(u, carry):
        for b in range(2):
            i = u * 2 + b

            @pl.when(i < nchunks_w)
            def _():
                wait(i, b)
                accv[...] += compute_chunk(xbufs[b], tbufs[b])

                @pl.when(i + 2 < nchunks_w)
                def _():
                    start(i + 2, b)
        return carry

    lax.fori_loop(0, _MAXC // 2, outer, 0)

    pltpu.sync_copy(accv, out_hbm.at[wid])


@jax.jit
def _rpn_cls_loss(x_in, t_in):
    x_blk = jnp.transpose(x_in.reshape(_NBLK, _BLK, 2), (0, 2, 1))
    mesh = plsc.VectorSubcoreMesh(core_axis_name="c", subcore_axis_name="s")
    partials = pl.kernel(
        _sc_body,
        out_type=jax.ShapeDtypeStruct((_NW, _LANES), jnp.float32),
        mesh=mesh,
        compiler_params=pltpu.CompilerParams(needs_layout_passes=False),
        scratch_types=[
            pltpu.VMEM((_CHUNK_B, 2, _BLK), jnp.float32),
            pltpu.VMEM((_CHUNK_B, 2, _BLK), jnp.float32),
            pltpu.VMEM((_CHUNK_E,), jnp.int32),
            pltpu.VMEM((_CHUNK_E,), jnp.int32),
            pltpu.VMEM((_LANES,), jnp.float32),
            pltpu.SemaphoreType.DMA,
            pltpu.SemaphoreType.DMA,
            pltpu.SemaphoreType.DMA,
            pltpu.SemaphoreType.DMA,
        ],
    )(x_blk, t_in)
    loss = jnp.sum(partials) * jnp.float32(1.0 / _N)
    return jnp.clip(loss, 0.0, 10.0)


def kernel(input, target):
    return _rpn_cls_loss(input, target)
